# baseline (speedup 1.0000x reference)
import functools

import jax
import jax.numpy as jnp
from jax import lax
from jax.experimental import pallas as pl
from jax.experimental.pallas import tpu as pltpu
from jax.experimental.pallas import tpu_sc as plsc


def _make_gather(B, H, V, D, NC, NS):
    NW = NC * NS
    EPW = B // NW
    CB = 2
    NBUF = 2
    NCH = EPW // CB

    mesh = plsc.VectorSubcoreMesh(core_axis_name="c", subcore_axis_name="s")

    @functools.partial(
        pl.kernel,
        mesh=mesh,
        out_type=jax.ShapeDtypeStruct((B, H, D), jnp.float32),
        scratch_types=[
            pltpu.VMEM((EPW * H,), jnp.int32),
            [pltpu.VMEM((CB * H, D), jnp.float32)] * NBUF,
            [pltpu.SemaphoreType.DMA] * NBUF,
            [pltpu.SemaphoreType.DMA] * NBUF,
        ],
        compiler_params=pltpu.CompilerParams(use_tc_tiling_on_sc=False),
    )
    def gather_kernel(idx_hbm, table_hbm, out_hbm, idx_v, rows, gsems, osems):
        wid = lax.axis_index("s") * NC + lax.axis_index("c")
        ebase = wid * EPW
        pltpu.sync_copy(idx_hbm.at[pl.ds(ebase * H, EPW * H)], idx_v)

        def gather_desc(j, b):
            return pltpu.make_async_copy(
                table_hbm.at[idx_v.at[pl.ds(j * CB * H, CB * H)]],
                rows[b],
                gsems[b],
            )

        def out_descs(j, b):
            return [
                pltpu.make_async_copy(
                    rows[b].at[pl.ds(k * H, H)],
                    out_hbm.at[ebase + j * CB + k],
                    osems[b],
                )
                for k in range(CB)
            ]

        for b in range(NBUF):
            gather_desc(b, b).start()

        def body(p, _):
            j0 = p * NBUF
            for b in range(NBUF):
                gather_desc(j0 + b, b).wait()
                for d in out_descs(j0 + b, b):
                    d.start()
            for b in range(NBUF):
                for d in out_descs(j0 + b, b):
                    d.wait()
                gather_desc(j0 + NBUF + b, b).start()
            return 0

        lax.fori_loop(0, NCH // NBUF - 1, body, 0)

        j0 = NCH - NBUF
        for b in range(NBUF):
            gather_desc(j0 + b, b).wait()
            for d in out_descs(j0 + b, b):
                d.start()
        for b in range(NBUF):
            for d in out_descs(j0 + b, b):
                d.wait()

    return gather_kernel


def kernel(indices, table):
    B, H = indices.shape
    V, D = table.shape
    flat_idx = indices.reshape(B * H).astype(jnp.int32)
    info = plsc.get_sparse_core_info()
    return _make_gather(B, H, V, D, info.num_cores, info.num_subcores)(
        flat_idx, table
    )
